# baseline (speedup 1.0000x reference)
import functools

import jax
import jax.numpy as jnp
from jax import lax
from jax.experimental import pallas as pl
from jax.experimental.pallas import tpu as pltpu
from jax.experimental.pallas import tpu_sc as plsc

B = 16384
D = 64
NC = 2
NS = 16
NW = NC * NS
BPW = B // NW
L = 16
EPS = 1e-5

_mesh = plsc.VectorSubcoreMesh(core_axis_name="c", subcore_axis_name="s")


@functools.partial(
    pl.kernel,
    mesh=_mesh,
    out_type=jax.ShapeDtypeStruct((B, D), jnp.float32),
    scratch_types=[
        pltpu.VMEM((BPW,), jnp.int32),
        pltpu.VMEM((BPW, D), jnp.float32),
        pltpu.VMEM((D,), jnp.float32),
        pltpu.VMEM((D,), jnp.float32),
        pltpu.SemaphoreType.DMA,
    ],
    compiler_params=pltpu.CompilerParams(skip_device_barrier=True),
)
def _embed_ln(ids_hbm, w_hbm, g_hbm, b_hbm, out_hbm, idx_v, rows_v, g_v, b_v, sem_a):
    wid = lax.axis_index("s") * NC + lax.axis_index("c")
    base = wid * BPW

    pltpu.sync_copy(ids_hbm.at[pl.ds(base, BPW)], idx_v)
    pltpu.sync_copy(g_hbm, g_v)
    pltpu.sync_copy(b_hbm, b_v)

    gs = [g_v[pl.ds(c * L, L)] for c in range(D // L)]
    bs = [b_v[pl.ds(c * L, L)] for c in range(D // L)]

    iota = lax.iota(jnp.int32, L)
    perms = [iota ^ sh for sh in (1, 2, 4, 8)]

    def hsum(v):
        for p in perms:
            v = v + v.at[p].get(mode="promise_in_bounds")
        return v

    def issue(g, carry):
        vec = idx_v[pl.ds(g * L, L)]
        for k in range(L):
            pltpu.async_copy(w_hbm.at[vec[k]], rows_v.at[g * L + k], sem_a)
        return carry

    lax.fori_loop(0, BPW // L, issue, 0)
    pltpu.make_async_copy(out_hbm.at[pl.ds(0, BPW)], rows_v, sem_a).wait()

    @plsc.parallel_loop(0, BPW, step=1, unroll=16)
    def body(r):
        vs = [rows_v[r, pl.ds(c * L, L)] for c in range(D // L)]
        s = (vs[0] + vs[1]) + (vs[2] + vs[3])
        mean = hsum(s) * (1.0 / D)
        ds_ = [v - mean for v in vs]
        sq = (ds_[0] * ds_[0] + ds_[1] * ds_[1]) + (ds_[2] * ds_[2] + ds_[3] * ds_[3])
        var = hsum(sq) * (1.0 / D)
        x = var + EPS
        xi = lax.bitcast_convert_type(x, jnp.int32)
        magic = jnp.full((L,), 0x5F3759DF, dtype=jnp.int32)
        y = lax.bitcast_convert_type(magic - (xi >> 1), jnp.float32)
        hx = x * -0.5
        for _ in range(3):
            y = y * (y * y * hx + 1.5)
        for c in range(D // L):
            rows_v[r, pl.ds(c * L, L)] = ds_[c] * y * gs[c] + bs[c]

    pltpu.sync_copy(rows_v, out_hbm.at[pl.ds(base, BPW)])


def kernel(input_ids, weight, gamma, beta):
    ids = input_ids.reshape(-1).astype(jnp.int32)
    out = _embed_ln(ids, weight, gamma, beta)
    return out.reshape(B, 1, D)


# speedup vs baseline: 1.0223x; 1.0223x over previous
import functools

import jax
import jax.numpy as jnp
from jax import lax
from jax.experimental import pallas as pl
from jax.experimental.pallas import tpu as pltpu
from jax.experimental.pallas import tpu_sc as plsc

B = 16384
D = 64
NC = 2
NS = 16
NW = NC * NS
BPW = B // NW
L = 16
EPS = 1e-5

_mesh = plsc.VectorSubcoreMesh(core_axis_name="c", subcore_axis_name="s")


@functools.partial(
    pl.kernel,
    mesh=_mesh,
    out_type=jax.ShapeDtypeStruct((B, D), jnp.float32),
    scratch_types=[
        pltpu.VMEM((BPW,), jnp.int32),
        pltpu.VMEM((BPW, D), jnp.float32),
        pltpu.VMEM((D,), jnp.float32),
        pltpu.VMEM((D,), jnp.float32),
        pltpu.SemaphoreType.DMA,
    ],
)
def _embed_ln(ids_hbm, w_hbm, g_hbm, b_hbm, out_hbm, idx_v, rows_v, g_v, b_v, sem):
    wid = lax.axis_index("s") * NC + lax.axis_index("c")
    base = wid * BPW

    pltpu.sync_copy(ids_hbm.at[pl.ds(base, BPW)], idx_v)
    pltpu.sync_copy(g_hbm, g_v)
    pltpu.sync_copy(b_hbm, b_v)

    def issue(g, carry):
        vec = idx_v[pl.ds(g * L, L)]
        for k in range(L):
            pltpu.async_copy(w_hbm.at[vec[k]], rows_v.at[g * L + k], sem)
        return carry

    lax.fori_loop(0, BPW // L, issue, 0)
    pltpu.make_async_copy(out_hbm.at[pl.ds(0, BPW)], rows_v, sem).wait()

    gs = [g_v[pl.ds(c * L, L)] for c in range(D // L)]
    bs = [b_v[pl.ds(c * L, L)] for c in range(D // L)]

    iota = lax.iota(jnp.int32, L)
    perms = [iota ^ sh for sh in (1, 2, 4, 8)]

    def hsum(v):
        for p in perms:
            v = v + v.at[p].get(mode="promise_in_bounds")
        return v

    @plsc.parallel_loop(0, BPW, step=1, unroll=8)
    def body(r):
        vs = [rows_v[r, pl.ds(c * L, L)] for c in range(D // L)]
        s = (vs[0] + vs[1]) + (vs[2] + vs[3])
        mean = hsum(s) * (1.0 / D)
        ds_ = [v - mean for v in vs]
        sq = (ds_[0] * ds_[0] + ds_[1] * ds_[1]) + (ds_[2] * ds_[2] + ds_[3] * ds_[3])
        var = hsum(sq) * (1.0 / D)
        x = var + EPS
        xi = lax.bitcast_convert_type(x, jnp.int32)
        magic = jnp.full((L,), 0x5F3759DF, dtype=jnp.int32)
        y = lax.bitcast_convert_type(magic - (xi >> 1), jnp.float32)
        hx = x * -0.5
        for _ in range(3):
            y = y * (y * y * hx + 1.5)
        for c in range(D // L):
            rows_v[r, pl.ds(c * L, L)] = ds_[c] * y * gs[c] + bs[c]

    pltpu.sync_copy(rows_v, out_hbm.at[pl.ds(base, BPW)])


def kernel(input_ids, weight, gamma, beta):
    ids = input_ids.reshape(-1).astype(jnp.int32)
    out = _embed_ln(ids, weight, gamma, beta)
    return out.reshape(B, 1, D)
